# baseline (speedup 1.0000x reference)
import functools

import jax
import jax.numpy as jnp
import numpy as np
from jax.experimental import pallas as pl
from jax.experimental.pallas import tpu as pltpu

K = 30
DILATION = 8
OUT_C = 10
INF = float('inf')
IBIG = 2**30


def _knn_kernel(bounds_ref, rstart_ref, rend_ref, sqrow_ref, featsA_ref,
                rhsB_ref, out_ref, *, C, K, R, D):
    g = pl.program_id(0)
    lo = bounds_ref[g, 0]
    hi = bounds_ref[g, 1]
    rstart = rstart_ref[...]
    rend = rend_ref[...]
    sqrow = sqrow_ref[...]
    rhs = rhsB_ref[...].astype(jnp.bfloat16)

    topd0 = jnp.full((32, R), INF, dtype=jnp.float32)
    topi0 = jnp.full((32, R), IBIG, dtype=jnp.int32)
    li32 = jax.lax.broadcasted_iota(jnp.int32, (32, R), 0)

    def chunk_body(j, carry):
        topd, topi = carry
        base = j * C
        lhs = featsA_ref[pl.ds(base, C), :]
        p2 = jax.lax.dot_general(
            lhs.astype(jnp.bfloat16), rhs, (((1,), (0,)), ((), ())),
            preferred_element_type=jnp.float32)
        sqcol = jax.lax.slice(lhs, (0, D), (C, D + 1))
        e = (sqcol + sqrow) + p2
        col_ids = base + jax.lax.broadcasted_iota(jnp.int32, (C, R), 0)
        valid = (col_ids >= rstart) & (col_ids < rend)
        e = jnp.where(valid, e, INF)

        def extract(t, carry2):
            e2, topd, topi = carry2
            m = jnp.min(e2, axis=0, keepdims=True)
            cand = jnp.where(e2 == m, col_ids, IBIG)
            mi = jnp.min(cand, axis=0, keepdims=True)
            e2 = jnp.where(col_ids == mi, INF, e2)
            p = jnp.sum((topd <= m).astype(jnp.int32), axis=0, keepdims=True)
            sh_d = pltpu.roll(topd, 1, 0)
            sh_i = pltpu.roll(topi, 1, 0)
            topd = jnp.where(li32 < p, topd,
                             jnp.where(li32 == p, m, sh_d))
            topi = jnp.where(li32 < p, topi,
                             jnp.where(li32 == p, mi, sh_i))
            return e2, topd, topi

        _, topd, topi = jax.lax.fori_loop(0, K, extract, (e, topd, topi))
        return topd, topi

    _, topi = jax.lax.fori_loop(lo, hi, chunk_body, (topd0, topi0))
    out_ref[...] = topi


def _knn_pallas(feats, batch_i32, k, *, C=512, R=128):
    n, d = feats.shape
    nblk = (n + R - 1) // R
    npad = nblk * R
    dpad = max(8, int(np.ceil((d + 1) / 8)) * 8)

    sq = jnp.sum(feats * feats, axis=1)
    featsA = jnp.concatenate(
        [feats, sq[:, None],
         jnp.zeros((n, dpad - d - 1), jnp.float32)], axis=1)
    featsA = jnp.pad(featsA, ((0, npad - n), (0, 0)))
    ncpad = ((npad + C - 1) // C) * C
    featsA = jnp.pad(featsA, ((0, ncpad - npad), (0, 0)))
    rhsB = jnp.concatenate(
        [-2.0 * feats.T,
         jnp.zeros((dpad - d, n), jnp.float32)], axis=0)
    rhsB = jnp.pad(rhsB, ((0, 0), (0, npad - n)))
    sqrow = jnp.pad(sq, (0, npad - n)).reshape(1, npad)

    rstart = jnp.searchsorted(batch_i32, batch_i32, side='left').astype(jnp.int32)
    rend = jnp.searchsorted(batch_i32, batch_i32, side='right').astype(jnp.int32)
    rstart = jnp.pad(rstart, (0, npad - n), constant_values=IBIG)
    rend = jnp.pad(rend, (0, npad - n), constant_values=0)
    lo = jnp.min(rstart.reshape(nblk, R), axis=1) // C
    hi = (jnp.max(rend.reshape(nblk, R), axis=1) + C - 1) // C
    lo = jnp.minimum(lo, hi)
    bounds = jnp.stack([lo, hi], axis=1).astype(jnp.int32)

    kern = functools.partial(_knn_kernel, C=C, K=k, R=R, D=d)
    out = pl.pallas_call(
        kern,
        grid=(nblk,),
        in_specs=[
            pl.BlockSpec(memory_space=pltpu.MemorySpace.SMEM),
            pl.BlockSpec((1, R), lambda g: (0, g)),
            pl.BlockSpec((1, R), lambda g: (0, g)),
            pl.BlockSpec((1, R), lambda g: (0, g)),
            pl.BlockSpec(memory_space=pltpu.MemorySpace.VMEM),
            pl.BlockSpec((dpad, R), lambda g: (0, g)),
        ],
        out_specs=pl.BlockSpec((32, R), lambda g: (0, g)),
        out_shape=jax.ShapeDtypeStruct((32, npad), jnp.int32),
    )(bounds, rstart.reshape(1, npad), rend.reshape(1, npad), sqrow,
      featsA, rhsB)
    return out[:k, :n]


NPAD = 10240
BN_EPS = 1e-5


def _gather_sc(table, idx_flat):
    from jax.experimental.pallas import tpu_sc as plsc

    m = idx_flat.shape[0]
    dp = table.shape[1]
    window = 128
    idx2 = idx_flat.reshape(1, m)
    mesh = plsc.VectorSubcoreMesh(core_axis_name="c", subcore_axis_name="s")

    @functools.partial(
        pl.kernel,
        out_type=jax.ShapeDtypeStruct((m, dp), table.dtype),
        mesh=mesh)
    def _gk(x_hbm, i_hbm, o_hbm):
        def body(i_vmem, o_vmem):
            pltpu.sync_copy(x_hbm.at[i_vmem.at[0]], o_vmem)

        pltpu.emit_pipeline(
            body,
            grid=(m // window,),
            in_specs=[pl.BlockSpec((1, window), lambda i: (0, i))],
            out_specs=[pl.BlockSpec((window, dp), lambda i: (i, 0))],
            core_axis_name=("c", "s"),
            dimension_semantics=(pltpu.PARALLEL,),
        )(i_hbm, o_hbm)

    return _gk(table, idx2)


def _edge_pass1_kernel(feats_ref, xj_ref, w1a_ref, b1a_ref, w1b_ref, b1b_ref,
                       h1a_ref, h1b_ref, stats_ref, *, NREAL, KRANK):
    r = pl.program_id(0)
    xi = feats_ref[...]
    dp = xi.shape[1]
    xj = xj_ref[...][:, :dp]
    a = jnp.concatenate([xi, xj - xi], axis=1).astype(jnp.bfloat16)
    npad = xi.shape[0]
    node = jax.lax.broadcasted_iota(jnp.int32, (npad, 1), 0)
    valid = node < NREAL
    dil = valid & (((node * KRANK + r) & 7) == 0)

    h1a = jnp.maximum(
        jax.lax.dot_general(a, w1a_ref[...], (((1,), (0,)), ((), ())),
                            preferred_element_type=jnp.float32)
        + b1a_ref[...], 0.0)
    h1b = jnp.maximum(
        jax.lax.dot_general(a, w1b_ref[...], (((1,), (0,)), ((), ())),
                            preferred_element_type=jnp.float32)
        + b1b_ref[...], 0.0)
    h1a_ref[...] = h1a
    h1b_ref[...] = h1b

    za = jnp.where(valid, h1a, 0.0)
    zb = jnp.where(dil, h1b, 0.0)
    part = jnp.concatenate([
        jnp.sum(za, axis=0, keepdims=True),
        jnp.sum(za * za, axis=0, keepdims=True),
        jnp.sum(zb, axis=0, keepdims=True),
        jnp.sum(zb * zb, axis=0, keepdims=True),
        jnp.zeros((4, za.shape[1]), jnp.float32)], axis=0)

    @pl.when(r == 0)
    def _():
        stats_ref[...] = jnp.zeros_like(stats_ref)

    stats_ref[...] += part


def _edge_pass2_kernel(h1a_ref, h1b_ref, stats1_ref, w2a_ref, b2a_ref,
                       g1a_ref, be1a_ref, w2b_ref, b2b_ref, g1b_ref,
                       be1b_ref, maxa_ref, mina_ref, maxb_ref, minb_ref,
                       stats2_ref, *, NREAL, KRANK, EA, EB):
    r = pl.program_id(0)
    npad = h1a_ref.shape[0]
    node = jax.lax.broadcasted_iota(jnp.int32, (npad, 1), 0)
    valid = node < NREAL
    dil = valid & (((node * KRANK + r) & 7) == 0)
    st = stats1_ref[...]

    def bn_apply(h, srow, qrow, e, g_ref, be_ref):
        mu = st[srow:srow + 1, :] / e
        var = st[qrow:qrow + 1, :] / e - mu * mu
        return (h - mu) / jnp.sqrt(var + BN_EPS) * g_ref[...] + be_ref[...]

    h2a = jnp.maximum(
        jax.lax.dot_general(
            bn_apply(h1a_ref[...], 0, 1, EA, g1a_ref, be1a_ref
                     ).astype(jnp.bfloat16),
            w2a_ref[...], (((1,), (0,)), ((), ())),
            preferred_element_type=jnp.float32) + b2a_ref[...], 0.0)
    h2b = jnp.maximum(
        jax.lax.dot_general(
            bn_apply(h1b_ref[...], 2, 3, EB, g1b_ref, be1b_ref
                     ).astype(jnp.bfloat16),
            w2b_ref[...], (((1,), (0,)), ((), ())),
            preferred_element_type=jnp.float32) + b2b_ref[...], 0.0)

    za = jnp.where(valid, h2a, 0.0)
    zb = jnp.where(dil, h2b, 0.0)
    part = jnp.concatenate([
        jnp.sum(za, axis=0, keepdims=True),
        jnp.sum(za * za, axis=0, keepdims=True),
        jnp.sum(zb, axis=0, keepdims=True),
        jnp.sum(zb * zb, axis=0, keepdims=True),
        jnp.zeros((4, za.shape[1]), jnp.float32)], axis=0)

    @pl.when(r == 0)
    def _():
        stats2_ref[...] = jnp.zeros_like(stats2_ref)
        maxa_ref[...] = jnp.full_like(maxa_ref, -INF)
        mina_ref[...] = jnp.full_like(mina_ref, INF)
        maxb_ref[...] = jnp.full_like(maxb_ref, -INF)
        minb_ref[...] = jnp.full_like(minb_ref, INF)

    stats2_ref[...] += part
    maxa_ref[...] = jnp.maximum(maxa_ref[...], h2a)
    mina_ref[...] = jnp.minimum(mina_ref[...], h2a)
    maxb_ref[...] = jnp.maximum(maxb_ref[...], jnp.where(dil, h2b, -INF))
    minb_ref[...] = jnp.minimum(minb_ref[...], jnp.where(dil, h2b, INF))


def _edge_conv_pair(feats, idxT, la, lb, n, k):
    npad, dp = feats.shape
    table = jnp.pad(feats, ((0, 0), (0, 128 - dp)))
    xj = _gather_sc(table, idxT.reshape(-1))

    d2 = 2 * dp

    def expand_w1(w, d):
        wt, wb = w[:d], w[d:]
        return jnp.concatenate([
            jnp.pad(wt, ((0, dp - d), (0, 0))),
            jnp.pad(wb, ((0, dp - d), (0, 0)))], axis=0).astype(jnp.bfloat16)

    d_real_a = la[0]['w'].shape[0] // 2
    d_real_b = lb[0]['w'].shape[0] // 2
    w1a = expand_w1(la[0]['w'], d_real_a)
    w1b = expand_w1(lb[0]['w'], d_real_b)

    row64 = lambda v: v.reshape(1, 64)
    kern1 = functools.partial(_edge_pass1_kernel, NREAL=n, KRANK=k)
    h1a, h1b, stats1 = pl.pallas_call(
        kern1,
        grid=(k,),
        in_specs=[
            pl.BlockSpec((npad, dp), lambda r: (0, 0)),
            pl.BlockSpec((npad, 128), lambda r: (r, 0)),
            pl.BlockSpec((d2, 64), lambda r: (0, 0)),
            pl.BlockSpec((1, 64), lambda r: (0, 0)),
            pl.BlockSpec((d2, 64), lambda r: (0, 0)),
            pl.BlockSpec((1, 64), lambda r: (0, 0)),
        ],
        out_specs=[
            pl.BlockSpec((npad, 64), lambda r: (r, 0)),
            pl.BlockSpec((npad, 64), lambda r: (r, 0)),
            pl.BlockSpec((8, 64), lambda r: (0, 0)),
        ],
        out_shape=[
            jax.ShapeDtypeStruct((k * npad, 64), jnp.float32),
            jax.ShapeDtypeStruct((k * npad, 64), jnp.float32),
            jax.ShapeDtypeStruct((8, 64), jnp.float32),
        ],
    )(feats, xj, w1a, row64(la[0]['b']),
      w1b, row64(lb[0]['b']))

    ea = float(n * k)
    eb = float((n * k + 7) // 8)
    kern2 = functools.partial(_edge_pass2_kernel, NREAL=n, KRANK=k,
                              EA=ea, EB=eb)
    w2a = la[1]['w'].astype(jnp.bfloat16)
    w2b = lb[1]['w'].astype(jnp.bfloat16)
    maxa, mina, maxb, minb, stats2 = pl.pallas_call(
        kern2,
        grid=(k,),
        in_specs=[
            pl.BlockSpec((npad, 64), lambda r: (r, 0)),
            pl.BlockSpec((npad, 64), lambda r: (r, 0)),
            pl.BlockSpec((8, 64), lambda r: (0, 0)),
            pl.BlockSpec((64, 64), lambda r: (0, 0)),
            pl.BlockSpec((1, 64), lambda r: (0, 0)),
            pl.BlockSpec((1, 64), lambda r: (0, 0)),
            pl.BlockSpec((1, 64), lambda r: (0, 0)),
            pl.BlockSpec((64, 64), lambda r: (0, 0)),
            pl.BlockSpec((1, 64), lambda r: (0, 0)),
            pl.BlockSpec((1, 64), lambda r: (0, 0)),
            pl.BlockSpec((1, 64), lambda r: (0, 0)),
        ],
        out_specs=[
            pl.BlockSpec((npad, 64), lambda r: (0, 0)),
            pl.BlockSpec((npad, 64), lambda r: (0, 0)),
            pl.BlockSpec((npad, 64), lambda r: (0, 0)),
            pl.BlockSpec((npad, 64), lambda r: (0, 0)),
            pl.BlockSpec((8, 64), lambda r: (0, 0)),
        ],
        out_shape=[
            jax.ShapeDtypeStruct((npad, 64), jnp.float32),
            jax.ShapeDtypeStruct((npad, 64), jnp.float32),
            jax.ShapeDtypeStruct((npad, 64), jnp.float32),
            jax.ShapeDtypeStruct((npad, 64), jnp.float32),
            jax.ShapeDtypeStruct((8, 64), jnp.float32),
        ],
    )(h1a, h1b, stats1, w2a, row64(la[1]['b']), row64(la[1]['g']),
      row64(la[1]['be']), w2b, row64(lb[1]['b']), row64(lb[1]['g']),
      row64(lb[1]['be']))
    return (maxa, mina, stats2[0:2], la[1]), (maxb, minb, stats2[2:4], lb[1])


def _head1_kernel(*refs, NREAL, ES):
    conv_refs = refs[:20]
    w_ref, b_ref, h_ref, stats_ref = refs[20:]
    i = pl.program_id(0)
    parts = []
    for c in range(4):
        mx, mn, st, g2, be2 = conv_refs[5 * c:5 * c + 5]
        mu = st[0:1, :] / ES[c]
        var = st[1:2, :] / ES[c] - mu * mu
        h = jnp.where(g2[...] > 0, mx[...], mn[...])
        parts.append((h - mu) / jnp.sqrt(var + BN_EPS) * g2[...] + be2[...])
    xcat = jnp.concatenate(parts, axis=1)
    pre = jax.lax.dot_general(
        xcat.astype(jnp.bfloat16), w_ref[...], (((1,), (0,)), ((), ())),
        preferred_element_type=jnp.float32) + b_ref[...]
    h = jnp.maximum(pre, 0.0)
    h_ref[...] = h
    bsz = h.shape[0]
    node = i * bsz + jax.lax.broadcasted_iota(jnp.int32, (bsz, 1), 0)
    z = jnp.where(node < NREAL, h, 0.0)
    part = jnp.concatenate([jnp.sum(z, axis=0, keepdims=True),
                            jnp.sum(z * z, axis=0, keepdims=True)], axis=0)

    @pl.when(i == 0)
    def _():
        stats_ref[...] = jnp.zeros_like(stats_ref)

    stats_ref[...] += part


def _head_mid_kernel(h_ref, st_ref, g_ref, be_ref, w_ref, b_ref,
                     o_ref, stats_ref, *, NREAL, E):
    i = pl.program_id(0)
    mu = st_ref[0:1, :] / E
    var = st_ref[1:2, :] / E - mu * mu
    hn = (h_ref[...] - mu) / jnp.sqrt(var + BN_EPS) * g_ref[...] + be_ref[...]
    pre = jax.lax.dot_general(
        hn.astype(jnp.bfloat16), w_ref[...], (((1,), (0,)), ((), ())),
        preferred_element_type=jnp.float32) + b_ref[...]
    h = jnp.maximum(pre, 0.0)
    o_ref[...] = h
    bsz = h.shape[0]
    node = i * bsz + jax.lax.broadcasted_iota(jnp.int32, (bsz, 1), 0)
    z = jnp.where(node < NREAL, h, 0.0)
    part = jnp.concatenate([jnp.sum(z, axis=0, keepdims=True),
                            jnp.sum(z * z, axis=0, keepdims=True)], axis=0)

    @pl.when(i == 0)
    def _():
        stats_ref[...] = jnp.zeros_like(stats_ref)

    stats_ref[...] += part


def _head_final_kernel(h_ref, st_ref, g_ref, be_ref, w_ref, b_ref, o_ref,
                       *, E, OUTC):
    mu = st_ref[0:1, :] / E
    var = st_ref[1:2, :] / E - mu * mu
    hn = (h_ref[...] - mu) / jnp.sqrt(var + BN_EPS) * g_ref[...] + be_ref[...]
    pre = jax.lax.dot_general(
        hn.astype(jnp.bfloat16), w_ref[...], (((1,), (0,)), ((), ())),
        preferred_element_type=jnp.float32) + b_ref[...]
    lane = jax.lax.broadcasted_iota(jnp.int32, pre.shape, 1)
    live = lane < OUTC
    pm = jnp.where(live, pre, -INF)
    mx = jnp.max(pm, axis=1, keepdims=True)
    sh = pre - mx
    ex = jnp.where(live, jnp.exp(sh), 0.0)
    lse = jnp.log(jnp.sum(ex, axis=1, keepdims=True))
    o_ref[...] = sh - lse


def kernel(x, pos, batch, params):
    n = x.shape[0]
    b32 = batch.astype(jnp.int32)
    idxT_pos = _knn_pallas(pos, b32, K)
    idxT_x = _knn_pallas(x, b32, K)

    def padded(feats, dp):
        d = feats.shape[1]
        return jnp.pad(feats, ((0, NPAD - n), (0, dp - d)))

    pos_p = padded(pos, 32)
    x_p = padded(x, 32)
    pad_idx = lambda t: jnp.pad(t, ((0, 0), (0, NPAD - n)))
    c1, c2 = _edge_conv_pair(pos_p, pad_idx(idxT_pos),
                             params['conv1'], params['conv2'], n, K)
    c3, c4 = _edge_conv_pair(x_p, pad_idx(idxT_x),
                             params['conv3'], params['conv4'], n, K)

    ne_full = float(n * K)
    ne_dil = float((n * K + DILATION - 1) // DILATION)
    ES = (ne_full, ne_dil, ne_full, ne_dil)

    B = 2048
    nb = NPAD // B
    row = lambda v: v.reshape(1, -1)
    conv_inputs = []
    conv_specs = []
    for (mx, mn, st, lyr) in (c1, c2, c3, c4):
        conv_inputs += [mx, mn, st, row(lyr['g']), row(lyr['be'])]
        conv_specs += [
            pl.BlockSpec((B, 64), lambda i: (i, 0)),
            pl.BlockSpec((B, 64), lambda i: (i, 0)),
            pl.BlockSpec((2, 64), lambda i: (0, 0)),
            pl.BlockSpec((1, 64), lambda i: (0, 0)),
            pl.BlockSpec((1, 64), lambda i: (0, 0)),
        ]

    l1 = params['lin1'][0]
    kern_h1 = functools.partial(_head1_kernel, NREAL=n, ES=ES)
    h1, st1 = pl.pallas_call(
        kern_h1,
        grid=(nb,),
        in_specs=conv_specs + [
            pl.BlockSpec((256, 1024), lambda i: (0, 0)),
            pl.BlockSpec((1, 1024), lambda i: (0, 0)),
        ],
        out_specs=[
            pl.BlockSpec((B, 1024), lambda i: (i, 0)),
            pl.BlockSpec((2, 1024), lambda i: (0, 0)),
        ],
        out_shape=[
            jax.ShapeDtypeStruct((NPAD, 1024), jnp.float32),
            jax.ShapeDtypeStruct((2, 1024), jnp.float32),
        ],
    )(*conv_inputs, l1['w'].astype(jnp.bfloat16), row(l1['b']))

    def mid(h, st, lyr_prev, lyr, din, dout):
        kern = functools.partial(_head_mid_kernel, NREAL=n, E=float(n))
        return pl.pallas_call(
            kern,
            grid=(nb,),
            in_specs=[
                pl.BlockSpec((B, din), lambda i: (i, 0)),
                pl.BlockSpec((2, din), lambda i: (0, 0)),
                pl.BlockSpec((1, din), lambda i: (0, 0)),
                pl.BlockSpec((1, din), lambda i: (0, 0)),
                pl.BlockSpec((din, dout), lambda i: (0, 0)),
                pl.BlockSpec((1, dout), lambda i: (0, 0)),
            ],
            out_specs=[
                pl.BlockSpec((B, dout), lambda i: (i, 0)),
                pl.BlockSpec((2, dout), lambda i: (0, 0)),
            ],
            out_shape=[
                jax.ShapeDtypeStruct((NPAD, dout), jnp.float32),
                jax.ShapeDtypeStruct((2, dout), jnp.float32),
            ],
        )(h, st, row(lyr_prev['g']), row(lyr_prev['be']),
          lyr['w'].astype(jnp.bfloat16), row(lyr['b']))

    h2, st2 = mid(h1, st1, l1, params['m1'][0], 1024, 256)
    h3, st3 = mid(h2, st2, params['m1'][0], params['m2'][0], 256, 128)

    m2l = params['m2'][0]
    wfin = jnp.pad(params['final_w'], ((0, 0), (0, 128 - OUT_C)))
    bfin = jnp.pad(params['final_b'], (0, 128 - OUT_C))
    kern_f = functools.partial(_head_final_kernel, E=float(n), OUTC=OUT_C)
    out = pl.pallas_call(
        kern_f,
        grid=(nb,),
        in_specs=[
            pl.BlockSpec((B, 128), lambda i: (i, 0)),
            pl.BlockSpec((2, 128), lambda i: (0, 0)),
            pl.BlockSpec((1, 128), lambda i: (0, 0)),
            pl.BlockSpec((1, 128), lambda i: (0, 0)),
            pl.BlockSpec((128, 128), lambda i: (0, 0)),
            pl.BlockSpec((1, 128), lambda i: (0, 0)),
        ],
        out_specs=pl.BlockSpec((B, 128), lambda i: (i, 0)),
        out_shape=jax.ShapeDtypeStruct((NPAD, 128), jnp.float32),
    )(h3, st3, row(m2l['g']), row(m2l['be']),
      wfin.astype(jnp.bfloat16), row(bfin))
    return out[:n, :OUT_C]


# speedup vs baseline: 1.1773x; 1.1773x over previous
import functools

import jax
import jax.numpy as jnp
import numpy as np
from jax.experimental import pallas as pl
from jax.experimental.pallas import tpu as pltpu

K = 30
DILATION = 8
OUT_C = 10
INF = float('inf')
IBIG = 2**30


def _knn_kernel(bounds_ref, rstart_ref, rend_ref, sqrow_ref, featsA_ref,
                rhsB_ref, out_ref, *, C, K, R, D):
    g = pl.program_id(0)
    lo = bounds_ref[g, 0]
    hi = bounds_ref[g, 1]
    rstart = rstart_ref[...]
    rend = rend_ref[...]
    sqrow = sqrow_ref[...]
    rhs = rhsB_ref[...].astype(jnp.bfloat16)

    topd0 = jnp.full((32, R), INF, dtype=jnp.float32)
    topi0 = jnp.full((32, R), IBIG, dtype=jnp.int32)
    li32 = jax.lax.broadcasted_iota(jnp.int32, (32, R), 0)

    def chunk_body(j, carry):
        topd, topi = carry
        base = j * C
        lhs = featsA_ref[pl.ds(base, C), :]
        p2 = jax.lax.dot_general(
            lhs.astype(jnp.bfloat16), rhs, (((1,), (0,)), ((), ())),
            preferred_element_type=jnp.float32)
        sqcol = jax.lax.slice(lhs, (0, D), (C, D + 1))
        e = (sqcol + sqrow) + p2
        col_ids = base + jax.lax.broadcasted_iota(jnp.int32, (C, R), 0)
        valid = (col_ids >= rstart) & (col_ids < rend)
        e = jnp.where(valid, e, INF)

        def extract(t, carry2):
            e2, topd, topi = carry2
            m = jnp.min(e2, axis=0, keepdims=True)
            cand = jnp.where(e2 == m, col_ids, IBIG)
            mi = jnp.min(cand, axis=0, keepdims=True)
            e2 = jnp.where(col_ids == mi, INF, e2)
            p = jnp.sum((topd <= m).astype(jnp.int32), axis=0, keepdims=True)
            sh_d = pltpu.roll(topd, 1, 0)
            sh_i = pltpu.roll(topi, 1, 0)
            topd = jnp.where(li32 < p, topd,
                             jnp.where(li32 == p, m, sh_d))
            topi = jnp.where(li32 < p, topi,
                             jnp.where(li32 == p, mi, sh_i))
            return e2, topd, topi

        _, topd, topi = jax.lax.fori_loop(0, K, extract, (e, topd, topi))
        return topd, topi

    _, topi = jax.lax.fori_loop(lo, hi, chunk_body, (topd0, topi0))
    out_ref[...] = topi


def _knn_pallas(feats, batch_i32, k, *, C=512, R=128):
    n, d = feats.shape
    nblk = (n + R - 1) // R
    npad = nblk * R
    dpad = max(8, int(np.ceil((d + 1) / 8)) * 8)

    sq = jnp.sum(feats * feats, axis=1)
    featsA = jnp.concatenate(
        [feats, sq[:, None],
         jnp.zeros((n, dpad - d - 1), jnp.float32)], axis=1)
    featsA = jnp.pad(featsA, ((0, npad - n), (0, 0)))
    ncpad = ((npad + C - 1) // C) * C
    featsA = jnp.pad(featsA, ((0, ncpad - npad), (0, 0)))
    rhsB = jnp.concatenate(
        [-2.0 * feats.T,
         jnp.zeros((dpad - d, n), jnp.float32)], axis=0)
    rhsB = jnp.pad(rhsB, ((0, 0), (0, npad - n)))
    sqrow = jnp.pad(sq, (0, npad - n)).reshape(1, npad)

    nb_vals = 8
    counts = jnp.sum(
        (batch_i32[None, :] == jnp.arange(nb_vals, dtype=jnp.int32)[:, None]
         ).astype(jnp.int32), axis=1)
    ccum = jnp.cumsum(counts)
    cstart = ccum - counts
    rstart = jnp.zeros_like(batch_i32)
    rend = jnp.zeros_like(batch_i32)
    for b in range(nb_vals):
        rstart = jnp.where(batch_i32 == b, cstart[b], rstart)
        rend = jnp.where(batch_i32 == b, ccum[b], rend)
    rstart = jnp.pad(rstart, (0, npad - n), constant_values=IBIG)
    rend = jnp.pad(rend, (0, npad - n), constant_values=0)
    lo = jnp.min(rstart.reshape(nblk, R), axis=1) // C
    hi = (jnp.max(rend.reshape(nblk, R), axis=1) + C - 1) // C
    lo = jnp.minimum(lo, hi)
    bounds = jnp.stack([lo, hi], axis=1).astype(jnp.int32)

    kern = functools.partial(_knn_kernel, C=C, K=k, R=R, D=d)
    out = pl.pallas_call(
        kern,
        grid=(nblk,),
        in_specs=[
            pl.BlockSpec(memory_space=pltpu.MemorySpace.SMEM),
            pl.BlockSpec((1, R), lambda g: (0, g)),
            pl.BlockSpec((1, R), lambda g: (0, g)),
            pl.BlockSpec((1, R), lambda g: (0, g)),
            pl.BlockSpec(memory_space=pltpu.MemorySpace.VMEM),
            pl.BlockSpec((dpad, R), lambda g: (0, g)),
        ],
        out_specs=pl.BlockSpec((32, R), lambda g: (0, g)),
        out_shape=jax.ShapeDtypeStruct((32, npad), jnp.int32),
    )(bounds, rstart.reshape(1, npad), rend.reshape(1, npad), sqrow,
      featsA, rhsB)
    return out[:k, :n]


NPAD = 10240
BN_EPS = 1e-5


def _gather_sc(table, idx_flat):
    from jax.experimental.pallas import tpu_sc as plsc

    m = idx_flat.shape[0]
    dp = table.shape[1]
    window = 128
    idx2 = idx_flat.reshape(1, m)
    mesh = plsc.VectorSubcoreMesh(core_axis_name="c", subcore_axis_name="s")

    @functools.partial(
        pl.kernel,
        out_type=jax.ShapeDtypeStruct((m, dp), table.dtype),
        mesh=mesh)
    def _gk(x_hbm, i_hbm, o_hbm):
        def body(i_vmem, o_vmem):
            pltpu.sync_copy(x_hbm.at[i_vmem.at[0]], o_vmem)

        pltpu.emit_pipeline(
            body,
            grid=(m // window,),
            in_specs=[pl.BlockSpec((1, window), lambda i: (0, i))],
            out_specs=[pl.BlockSpec((window, dp), lambda i: (i, 0))],
            core_axis_name=("c", "s"),
            dimension_semantics=(pltpu.PARALLEL,),
        )(i_hbm, o_hbm)

    return _gk(table, idx2)


def _edge_pass1_kernel(feats_ref, xj_ref, w1a_ref, b1a_ref, w1b_ref, b1b_ref,
                       h1a_ref, h1b_ref, stats_ref, *, NREAL, KRANK):
    r = pl.program_id(0)
    xi = feats_ref[...]
    dp = xi.shape[1]
    xj = xj_ref[...][:, :dp]
    a = jnp.concatenate([xi, xj - xi], axis=1).astype(jnp.bfloat16)
    npad = xi.shape[0]
    node = jax.lax.broadcasted_iota(jnp.int32, (npad, 1), 0)
    valid = node < NREAL
    dil = valid & (((node * KRANK + r) & 7) == 0)

    h1a = jnp.maximum(
        jax.lax.dot_general(a, w1a_ref[...], (((1,), (0,)), ((), ())),
                            preferred_element_type=jnp.float32)
        + b1a_ref[...], 0.0)
    h1b = jnp.maximum(
        jax.lax.dot_general(a, w1b_ref[...], (((1,), (0,)), ((), ())),
                            preferred_element_type=jnp.float32)
        + b1b_ref[...], 0.0)
    h1a_ref[...] = h1a
    h1b_ref[...] = h1b

    za = jnp.where(valid, h1a, 0.0)
    zb = jnp.where(dil, h1b, 0.0)
    part = jnp.concatenate([
        jnp.sum(za, axis=0, keepdims=True),
        jnp.sum(za * za, axis=0, keepdims=True),
        jnp.sum(zb, axis=0, keepdims=True),
        jnp.sum(zb * zb, axis=0, keepdims=True),
        jnp.zeros((4, za.shape[1]), jnp.float32)], axis=0)

    @pl.when(r == 0)
    def _():
        stats_ref[...] = jnp.zeros_like(stats_ref)

    stats_ref[...] += part


def _edge_pass2_kernel(h1a_ref, h1b_ref, stats1_ref, w2a_ref, b2a_ref,
                       g1a_ref, be1a_ref, w2b_ref, b2b_ref, g1b_ref,
                       be1b_ref, maxa_ref, mina_ref, maxb_ref, minb_ref,
                       stats2_ref, *, NREAL, KRANK, EA, EB):
    r = pl.program_id(0)
    npad = h1a_ref.shape[0]
    node = jax.lax.broadcasted_iota(jnp.int32, (npad, 1), 0)
    valid = node < NREAL
    dil = valid & (((node * KRANK + r) & 7) == 0)
    st = stats1_ref[...]

    def bn_apply(h, srow, qrow, e, g_ref, be_ref):
        mu = st[srow:srow + 1, :] / e
        var = st[qrow:qrow + 1, :] / e - mu * mu
        return (h - mu) / jnp.sqrt(var + BN_EPS) * g_ref[...] + be_ref[...]

    h2a = jnp.maximum(
        jax.lax.dot_general(
            bn_apply(h1a_ref[...], 0, 1, EA, g1a_ref, be1a_ref
                     ).astype(jnp.bfloat16),
            w2a_ref[...], (((1,), (0,)), ((), ())),
            preferred_element_type=jnp.float32) + b2a_ref[...], 0.0)
    h2b = jnp.maximum(
        jax.lax.dot_general(
            bn_apply(h1b_ref[...], 2, 3, EB, g1b_ref, be1b_ref
                     ).astype(jnp.bfloat16),
            w2b_ref[...], (((1,), (0,)), ((), ())),
            preferred_element_type=jnp.float32) + b2b_ref[...], 0.0)

    za = jnp.where(valid, h2a, 0.0)
    zb = jnp.where(dil, h2b, 0.0)
    part = jnp.concatenate([
        jnp.sum(za, axis=0, keepdims=True),
        jnp.sum(za * za, axis=0, keepdims=True),
        jnp.sum(zb, axis=0, keepdims=True),
        jnp.sum(zb * zb, axis=0, keepdims=True),
        jnp.zeros((4, za.shape[1]), jnp.float32)], axis=0)

    @pl.when(r == 0)
    def _():
        stats2_ref[...] = jnp.zeros_like(stats2_ref)
        maxa_ref[...] = jnp.full_like(maxa_ref, -INF)
        mina_ref[...] = jnp.full_like(mina_ref, INF)
        maxb_ref[...] = jnp.full_like(maxb_ref, -INF)
        minb_ref[...] = jnp.full_like(minb_ref, INF)

    stats2_ref[...] += part
    maxa_ref[...] = jnp.maximum(maxa_ref[...], h2a)
    mina_ref[...] = jnp.minimum(mina_ref[...], h2a)
    maxb_ref[...] = jnp.maximum(maxb_ref[...], jnp.where(dil, h2b, -INF))
    minb_ref[...] = jnp.minimum(minb_ref[...], jnp.where(dil, h2b, INF))


def _edge_conv_pair(feats, idxT, la, lb, n, k):
    npad, dp = feats.shape
    table = jnp.pad(feats, ((0, 0), (0, 128 - dp)))
    xj = _gather_sc(table, idxT.reshape(-1))

    d2 = 2 * dp

    def expand_w1(w, d):
        wt, wb = w[:d], w[d:]
        return jnp.concatenate([
            jnp.pad(wt, ((0, dp - d), (0, 0))),
            jnp.pad(wb, ((0, dp - d), (0, 0)))], axis=0).astype(jnp.bfloat16)

    d_real_a = la[0]['w'].shape[0] // 2
    d_real_b = lb[0]['w'].shape[0] // 2
    w1a = expand_w1(la[0]['w'], d_real_a)
    w1b = expand_w1(lb[0]['w'], d_real_b)

    row64 = lambda v: v.reshape(1, 64)
    kern1 = functools.partial(_edge_pass1_kernel, NREAL=n, KRANK=k)
    h1a, h1b, stats1 = pl.pallas_call(
        kern1,
        grid=(k,),
        in_specs=[
            pl.BlockSpec((npad, dp), lambda r: (0, 0)),
            pl.BlockSpec((npad, 128), lambda r: (r, 0)),
            pl.BlockSpec((d2, 64), lambda r: (0, 0)),
            pl.BlockSpec((1, 64), lambda r: (0, 0)),
            pl.BlockSpec((d2, 64), lambda r: (0, 0)),
            pl.BlockSpec((1, 64), lambda r: (0, 0)),
        ],
        out_specs=[
            pl.BlockSpec((npad, 64), lambda r: (r, 0)),
            pl.BlockSpec((npad, 64), lambda r: (r, 0)),
            pl.BlockSpec((8, 64), lambda r: (0, 0)),
        ],
        out_shape=[
            jax.ShapeDtypeStruct((k * npad, 64), jnp.float32),
            jax.ShapeDtypeStruct((k * npad, 64), jnp.float32),
            jax.ShapeDtypeStruct((8, 64), jnp.float32),
        ],
    )(feats, xj, w1a, row64(la[0]['b']),
      w1b, row64(lb[0]['b']))

    ea = float(n * k)
    eb = float((n * k + 7) // 8)
    kern2 = functools.partial(_edge_pass2_kernel, NREAL=n, KRANK=k,
                              EA=ea, EB=eb)
    w2a = la[1]['w'].astype(jnp.bfloat16)
    w2b = lb[1]['w'].astype(jnp.bfloat16)
    maxa, mina, maxb, minb, stats2 = pl.pallas_call(
        kern2,
        grid=(k,),
        in_specs=[
            pl.BlockSpec((npad, 64), lambda r: (r, 0)),
            pl.BlockSpec((npad, 64), lambda r: (r, 0)),
            pl.BlockSpec((8, 64), lambda r: (0, 0)),
            pl.BlockSpec((64, 64), lambda r: (0, 0)),
            pl.BlockSpec((1, 64), lambda r: (0, 0)),
            pl.BlockSpec((1, 64), lambda r: (0, 0)),
            pl.BlockSpec((1, 64), lambda r: (0, 0)),
            pl.BlockSpec((64, 64), lambda r: (0, 0)),
            pl.BlockSpec((1, 64), lambda r: (0, 0)),
            pl.BlockSpec((1, 64), lambda r: (0, 0)),
            pl.BlockSpec((1, 64), lambda r: (0, 0)),
        ],
        out_specs=[
            pl.BlockSpec((npad, 64), lambda r: (0, 0)),
            pl.BlockSpec((npad, 64), lambda r: (0, 0)),
            pl.BlockSpec((npad, 64), lambda r: (0, 0)),
            pl.BlockSpec((npad, 64), lambda r: (0, 0)),
            pl.BlockSpec((8, 64), lambda r: (0, 0)),
        ],
        out_shape=[
            jax.ShapeDtypeStruct((npad, 64), jnp.float32),
            jax.ShapeDtypeStruct((npad, 64), jnp.float32),
            jax.ShapeDtypeStruct((npad, 64), jnp.float32),
            jax.ShapeDtypeStruct((npad, 64), jnp.float32),
            jax.ShapeDtypeStruct((8, 64), jnp.float32),
        ],
    )(h1a, h1b, stats1, w2a, row64(la[1]['b']), row64(la[1]['g']),
      row64(la[1]['be']), w2b, row64(lb[1]['b']), row64(lb[1]['g']),
      row64(lb[1]['be']))
    return (maxa, mina, stats2[0:2], la[1]), (maxb, minb, stats2[2:4], lb[1])


def _head1_kernel(*refs, NREAL, ES):
    conv_refs = refs[:20]
    w_ref, b_ref, h_ref, stats_ref = refs[20:]
    i = pl.program_id(0)
    parts = []
    for c in range(4):
        mx, mn, st, g2, be2 = conv_refs[5 * c:5 * c + 5]
        mu = st[0:1, :] / ES[c]
        var = st[1:2, :] / ES[c] - mu * mu
        h = jnp.where(g2[...] > 0, mx[...], mn[...])
        parts.append((h - mu) / jnp.sqrt(var + BN_EPS) * g2[...] + be2[...])
    xcat = jnp.concatenate(parts, axis=1)
    pre = jax.lax.dot_general(
        xcat.astype(jnp.bfloat16), w_ref[...], (((1,), (0,)), ((), ())),
        preferred_element_type=jnp.float32) + b_ref[...]
    h = jnp.maximum(pre, 0.0)
    h_ref[...] = h
    bsz = h.shape[0]
    node = i * bsz + jax.lax.broadcasted_iota(jnp.int32, (bsz, 1), 0)
    z = jnp.where(node < NREAL, h, 0.0)
    part = jnp.concatenate([jnp.sum(z, axis=0, keepdims=True),
                            jnp.sum(z * z, axis=0, keepdims=True)], axis=0)

    @pl.when(i == 0)
    def _():
        stats_ref[...] = jnp.zeros_like(stats_ref)

    stats_ref[...] += part


def _head_mid_kernel(h_ref, st_ref, g_ref, be_ref, w_ref, b_ref,
                     o_ref, stats_ref, *, NREAL, E):
    i = pl.program_id(0)
    mu = st_ref[0:1, :] / E
    var = st_ref[1:2, :] / E - mu * mu
    hn = (h_ref[...] - mu) / jnp.sqrt(var + BN_EPS) * g_ref[...] + be_ref[...]
    pre = jax.lax.dot_general(
        hn.astype(jnp.bfloat16), w_ref[...], (((1,), (0,)), ((), ())),
        preferred_element_type=jnp.float32) + b_ref[...]
    h = jnp.maximum(pre, 0.0)
    o_ref[...] = h
    bsz = h.shape[0]
    node = i * bsz + jax.lax.broadcasted_iota(jnp.int32, (bsz, 1), 0)
    z = jnp.where(node < NREAL, h, 0.0)
    part = jnp.concatenate([jnp.sum(z, axis=0, keepdims=True),
                            jnp.sum(z * z, axis=0, keepdims=True)], axis=0)

    @pl.when(i == 0)
    def _():
        stats_ref[...] = jnp.zeros_like(stats_ref)

    stats_ref[...] += part


def _head_final_kernel(h_ref, st_ref, g_ref, be_ref, w_ref, b_ref, o_ref,
                       *, E, OUTC):
    mu = st_ref[0:1, :] / E
    var = st_ref[1:2, :] / E - mu * mu
    hn = (h_ref[...] - mu) / jnp.sqrt(var + BN_EPS) * g_ref[...] + be_ref[...]
    pre = jax.lax.dot_general(
        hn.astype(jnp.bfloat16), w_ref[...], (((1,), (0,)), ((), ())),
        preferred_element_type=jnp.float32) + b_ref[...]
    lane = jax.lax.broadcasted_iota(jnp.int32, pre.shape, 1)
    live = lane < OUTC
    pm = jnp.where(live, pre, -INF)
    mx = jnp.max(pm, axis=1, keepdims=True)
    sh = pre - mx
    ex = jnp.where(live, jnp.exp(sh), 0.0)
    lse = jnp.log(jnp.sum(ex, axis=1, keepdims=True))
    o_ref[...] = sh - lse


def kernel(x, pos, batch, params):
    n = x.shape[0]
    b32 = batch.astype(jnp.int32)
    idxT_pos = _knn_pallas(pos, b32, K)
    idxT_x = _knn_pallas(x, b32, K)

    def padded(feats, dp):
        d = feats.shape[1]
        return jnp.pad(feats, ((0, NPAD - n), (0, dp - d)))

    pos_p = padded(pos, 32)
    x_p = padded(x, 32)
    pad_idx = lambda t: jnp.pad(t, ((0, 0), (0, NPAD - n)))
    c1, c2 = _edge_conv_pair(pos_p, pad_idx(idxT_pos),
                             params['conv1'], params['conv2'], n, K)
    c3, c4 = _edge_conv_pair(x_p, pad_idx(idxT_x),
                             params['conv3'], params['conv4'], n, K)

    ne_full = float(n * K)
    ne_dil = float((n * K + DILATION - 1) // DILATION)
    ES = (ne_full, ne_dil, ne_full, ne_dil)

    B = 2048
    nb = NPAD // B
    row = lambda v: v.reshape(1, -1)
    conv_inputs = []
    conv_specs = []
    for (mx, mn, st, lyr) in (c1, c2, c3, c4):
        conv_inputs += [mx, mn, st, row(lyr['g']), row(lyr['be'])]
        conv_specs += [
            pl.BlockSpec((B, 64), lambda i: (i, 0)),
            pl.BlockSpec((B, 64), lambda i: (i, 0)),
            pl.BlockSpec((2, 64), lambda i: (0, 0)),
            pl.BlockSpec((1, 64), lambda i: (0, 0)),
            pl.BlockSpec((1, 64), lambda i: (0, 0)),
        ]

    l1 = params['lin1'][0]
    kern_h1 = functools.partial(_head1_kernel, NREAL=n, ES=ES)
    h1, st1 = pl.pallas_call(
        kern_h1,
        grid=(nb,),
        in_specs=conv_specs + [
            pl.BlockSpec((256, 1024), lambda i: (0, 0)),
            pl.BlockSpec((1, 1024), lambda i: (0, 0)),
        ],
        out_specs=[
            pl.BlockSpec((B, 1024), lambda i: (i, 0)),
            pl.BlockSpec((2, 1024), lambda i: (0, 0)),
        ],
        out_shape=[
            jax.ShapeDtypeStruct((NPAD, 1024), jnp.float32),
            jax.ShapeDtypeStruct((2, 1024), jnp.float32),
        ],
    )(*conv_inputs, l1['w'].astype(jnp.bfloat16), row(l1['b']))

    def mid(h, st, lyr_prev, lyr, din, dout):
        kern = functools.partial(_head_mid_kernel, NREAL=n, E=float(n))
        return pl.pallas_call(
            kern,
            grid=(nb,),
            in_specs=[
                pl.BlockSpec((B, din), lambda i: (i, 0)),
                pl.BlockSpec((2, din), lambda i: (0, 0)),
                pl.BlockSpec((1, din), lambda i: (0, 0)),
                pl.BlockSpec((1, din), lambda i: (0, 0)),
                pl.BlockSpec((din, dout), lambda i: (0, 0)),
                pl.BlockSpec((1, dout), lambda i: (0, 0)),
            ],
            out_specs=[
                pl.BlockSpec((B, dout), lambda i: (i, 0)),
                pl.BlockSpec((2, dout), lambda i: (0, 0)),
            ],
            out_shape=[
                jax.ShapeDtypeStruct((NPAD, dout), jnp.float32),
                jax.ShapeDtypeStruct((2, dout), jnp.float32),
            ],
        )(h, st, row(lyr_prev['g']), row(lyr_prev['be']),
          lyr['w'].astype(jnp.bfloat16), row(lyr['b']))

    h2, st2 = mid(h1, st1, l1, params['m1'][0], 1024, 256)
    h3, st3 = mid(h2, st2, params['m1'][0], params['m2'][0], 256, 128)

    m2l = params['m2'][0]
    wfin = jnp.pad(params['final_w'], ((0, 0), (0, 128 - OUT_C)))
    bfin = jnp.pad(params['final_b'], (0, 128 - OUT_C))
    kern_f = functools.partial(_head_final_kernel, E=float(n), OUTC=OUT_C)
    out = pl.pallas_call(
        kern_f,
        grid=(nb,),
        in_specs=[
            pl.BlockSpec((B, 128), lambda i: (i, 0)),
            pl.BlockSpec((2, 128), lambda i: (0, 0)),
            pl.BlockSpec((1, 128), lambda i: (0, 0)),
            pl.BlockSpec((1, 128), lambda i: (0, 0)),
            pl.BlockSpec((128, 128), lambda i: (0, 0)),
            pl.BlockSpec((1, 128), lambda i: (0, 0)),
        ],
        out_specs=pl.BlockSpec((B, 128), lambda i: (i, 0)),
        out_shape=jax.ShapeDtypeStruct((NPAD, 128), jnp.float32),
    )(h3, st3, row(m2l['g']), row(m2l['be']),
      wfin.astype(jnp.bfloat16), row(bfin))
    return out[:n, :OUT_C]
